# baseline (speedup 1.0000x reference)
import jax
import jax.numpy as jnp
from jax import lax
from jax.experimental import pallas as pl
from jax.experimental.pallas import tpu as pltpu
from jax.experimental.pallas import tpu_sc as plsc

NC, NS, L = 2, 16, 16
NW = NC * NS
NPLANE, NROW, NCOL = 32, 512, 512
N = NPLANE * NROW * NCOL
NT = N // NW
R = 8
CH = R * NCOL
NCHUNK = NROW // R
NBUF = 4
HB = 512
NBKT = 2 * HB
HBF = float(HB)
HW = NBKT * L
QR, QC = HW // 128, 128


def _stage1_kernel(p_hbm, t_hbm, cnt_hbm, sum_hbm,
                   bufp, buft, hcnt, hsum,
                   semp0, semp1, semp2, semp3, semt0, semt1, semt2, semt3):
  wid = lax.axis_index("s") * NC + lax.axis_index("c")
  semp = (semp0, semp1, semp2, semp3)
  semt = (semt0, semt1, semt2, semt3)

  lane = lax.broadcasted_iota(jnp.int32, (L,), 0)
  ones = jnp.full((L,), 1.0, dtype=jnp.float32)
  zeros = jnp.zeros((L,), dtype=jnp.float32)

  @plsc.parallel_loop(0, HW // L, unroll=8)
  def _(j):
    hcnt[pl.ds(j * L, L)] = zeros
    hsum[pl.ds(j * L, L)] = zeros

  def start(c, b):
    pltpu.async_copy(p_hbm.at[wid, pl.ds(c * R, R), :],
                     bufp.at[pl.ds(b * R, R), :], semp[b])
    pltpu.async_copy(t_hbm.at[wid, pl.ds(c * R, R), :],
                     buft.at[pl.ds(b * R, R), :], semt[b])

  def wait(c, b):
    pltpu.make_async_copy(
        p_hbm.at[wid, pl.ds(c * R, R), :],
        bufp.at[pl.ds(b * R, R), :], semp[b]).wait()
    pltpu.make_async_copy(
        t_hbm.at[wid, pl.ds(c * R, R), :],
        buft.at[pl.ds(b * R, R), :], semt[b]).wait()

  for b in range(NBUF):
    start(b, b)

  def process(b):
    @plsc.parallel_loop(0, NCOL // L, unroll=4)
    def _(c):
      for s in range(R):
        p = bufp[b * R + s, pl.ds(c * L, L)]
        t = buft[b * R + s, pl.ds(c * L, L)]
        f = jnp.minimum(p * HBF, HBF - 1.0) + t * HBF
        idx = f.astype(jnp.int32) * L + lane
        plsc.addupdate_scatter(hcnt, [idx], ones)
        plsc.addupdate_scatter(hsum, [idx], p)

  def chunk_body(o, _):
    for b in range(NBUF):
      c = NBUF * o + b
      wait(c, b)
      process(b)

      @pl.when(c + NBUF < NCHUNK)
      def _():
        start(c + NBUF, b)
    return 0

  lax.fori_loop(0, NCHUNK // NBUF, chunk_body, 0)

  pltpu.sync_copy(hcnt, cnt_hbm.at[pl.ds(wid * HW, HW)])
  pltpu.sync_copy(hsum, sum_hbm.at[pl.ds(wid * HW, HW)])


def _stage1(p, t):
  mesh = plsc.VectorSubcoreMesh(
      core_axis_name="c", subcore_axis_name="s", num_cores=NC, num_subcores=NS)
  return pl.kernel(
      _stage1_kernel,
      out_type=(
          jax.ShapeDtypeStruct((NW * HW,), jnp.float32),
          jax.ShapeDtypeStruct((NW * HW,), jnp.float32),
      ),
      mesh=mesh,
      compiler_params=pltpu.CompilerParams(
          needs_layout_passes=False, use_tc_tiling_on_sc=True),
      scratch_types=[
          pltpu.VMEM((NBUF * R, NCOL), jnp.float32),
          pltpu.VMEM((NBUF * R, NCOL), jnp.float32),
          pltpu.VMEM((HW,), jnp.float32),
          pltpu.VMEM((HW,), jnp.float32),
          pltpu.SemaphoreType.DMA,
          pltpu.SemaphoreType.DMA,
          pltpu.SemaphoreType.DMA,
          pltpu.SemaphoreType.DMA,
          pltpu.SemaphoreType.DMA,
          pltpu.SemaphoreType.DMA,
          pltpu.SemaphoreType.DMA,
          pltpu.SemaphoreType.DMA,
      ],
  )(p, t)


def _stage2_kernel(c_ref, s_ref, o_ref):
  ch = jnp.sum(c_ref[...].reshape(NW, QR, QC), axis=0)
  sh = jnp.sum(s_ref[...].reshape(NW, QR, QC), axis=0)

  colq = lax.broadcasted_iota(jnp.int32, (QC, 8), 0) // L
  qid = lax.broadcasted_iota(jnp.int32, (QC, 8), 1)
  m16 = (colq == qid).astype(jnp.float32)
  c8 = jnp.dot(ch, m16, preferred_element_type=jnp.float32)
  s8 = jnp.dot(sh, m16, preferred_element_type=jnp.float32)
  rneg = HB // 8

  npos = jnp.sum(c8[rneg:, :])
  s_pos = jnp.sum(s8[rneg:, :])
  cn = c8[:rneg, :]
  sn = s8[:rneg, :]
  nneg = jnp.sum(cn)
  s_neg = jnp.sum(sn)

  negative_num = jnp.minimum(nneg, npos * 3.0)
  k = negative_num.astype(jnp.int32)
  kf = k.astype(jnp.float32)

  q1 = lax.broadcasted_iota(jnp.int32, (8, 8), 0)
  q2 = lax.broadcasted_iota(jnp.int32, (8, 8), 1)
  u8 = (q1 > q2).astype(jnp.float32)
  r1 = lax.broadcasted_iota(jnp.int32, (rneg, rneg), 0)
  r2 = lax.broadcasted_iota(jnp.int32, (rneg, rneg), 1)
  lo = (r1 < r2).astype(jnp.float32)

  rt_c = jnp.sum(cn, axis=1, keepdims=True)
  rt_s = jnp.sum(sn, axis=1, keepdims=True)
  f = jnp.dot(lo, rt_c, preferred_element_type=jnp.float32) + jnp.dot(
      cn, u8, preferred_element_type=jnp.float32)
  g = jnp.dot(lo, rt_s, preferred_element_type=jnp.float32) + jnp.dot(
      sn, u8, preferred_element_type=jnp.float32)

  sel = (f < kf) & (f + cn >= kf)
  ratio = sn / jnp.maximum(cn, 1.0)
  part = jnp.sum(jnp.where(sel, g + (kf - f) * ratio, 0.0))
  s_topk = jnp.where(kf >= nneg, s_neg, part)

  union = npos + s_pos + s_topk
  iou = 2.0 * s_pos / union
  o_ref[0] = 1.0 - iou
  o_ref[1] = iou


def _stage2(cnt, sm):
  return pl.pallas_call(
      _stage2_kernel,
      out_shape=jax.ShapeDtypeStruct((2,), jnp.float32),
      in_specs=[
          pl.BlockSpec(memory_space=pltpu.VMEM),
          pl.BlockSpec(memory_space=pltpu.VMEM),
      ],
      out_specs=pl.BlockSpec(memory_space=pltpu.SMEM),
  )(cnt, sm)


@jax.jit
def kernel(predicted, target, training_mask):
  del training_mask
  cnt, sm = _stage1(predicted, target)
  out = _stage2(cnt.reshape(NW * QR, QC), sm.reshape(NW * QR, QC))
  return (out[0], out[1])


# speedup vs baseline: 1.2969x; 1.2969x over previous
import jax
import jax.numpy as jnp
from jax import lax
from jax.experimental import pallas as pl
from jax.experimental.pallas import tpu as pltpu
from jax.experimental.pallas import tpu_sc as plsc

NC, NS, L = 2, 16, 16
NW = NC * NS
NPLANE, NROW, NCOL = 32, 512, 512
N = NPLANE * NROW * NCOL
NT = N // NW
R = 8
CH = R * NCOL
NCHUNK = NROW // R
NBUF = 2
HB = 512
NBKT = 2 * HB
HBF = float(HB)
HW = NBKT * L
QR, QC = HW // 128, 128


def _stage1_kernel(p_hbm, t_hbm, cnt_hbm, sum_hbm,
                   bufp, buft, hcnt, hsum,
                   semp0, semp1, semp2, semp3, semt0, semt1, semt2, semt3):
  wid = lax.axis_index("s") * NC + lax.axis_index("c")
  semp = (semp0, semp1, semp2, semp3)
  semt = (semt0, semt1, semt2, semt3)

  lane = lax.broadcasted_iota(jnp.int32, (L,), 0)
  ones = jnp.full((L,), 1.0, dtype=jnp.float32)
  zeros = jnp.zeros((L,), dtype=jnp.float32)

  @plsc.parallel_loop(0, HW // L, unroll=8)
  def _(j):
    hcnt[pl.ds(j * L, L)] = zeros
    hsum[pl.ds(j * L, L)] = zeros

  def start(c, b):
    pltpu.async_copy(p_hbm.at[wid, pl.ds(c * R, R), :],
                     bufp.at[pl.ds(b * R, R), :], semp[b])
    pltpu.async_copy(t_hbm.at[wid, pl.ds(c * R, R), :],
                     buft.at[pl.ds(b * R, R), :], semt[b])

  def wait(c, b):
    pltpu.make_async_copy(
        p_hbm.at[wid, pl.ds(c * R, R), :],
        bufp.at[pl.ds(b * R, R), :], semp[b]).wait()
    pltpu.make_async_copy(
        t_hbm.at[wid, pl.ds(c * R, R), :],
        buft.at[pl.ds(b * R, R), :], semt[b]).wait()

  for b in range(NBUF):
    start(b, b)

  def process(b):
    @plsc.parallel_loop(0, NCOL // L, unroll=4)
    def _(c):
      for s in range(R):
        p = bufp[b * R + s, pl.ds(c * L, L)]
        t = buft[b * R + s, pl.ds(c * L, L)]
        f = jnp.minimum(p * HBF, HBF - 1.0) + t * HBF
        idx = f.astype(jnp.int32) * L + lane
        plsc.addupdate_scatter(hcnt, [idx], ones)
        plsc.addupdate_scatter(hsum, [idx], p)

  def chunk_body(o, _):
    for b in range(NBUF):
      c = NBUF * o + b
      wait(c, b)
      process(b)

      @pl.when(c + NBUF < NCHUNK)
      def _():
        start(c + NBUF, b)
    return 0

  lax.fori_loop(0, NCHUNK // NBUF, chunk_body, 0)

  pltpu.sync_copy(hcnt, cnt_hbm.at[pl.ds(wid * HW, HW)])
  pltpu.sync_copy(hsum, sum_hbm.at[pl.ds(wid * HW, HW)])


def _stage1(p, t):
  mesh = plsc.VectorSubcoreMesh(
      core_axis_name="c", subcore_axis_name="s", num_cores=NC, num_subcores=NS)
  return pl.kernel(
      _stage1_kernel,
      out_type=(
          jax.ShapeDtypeStruct((NW * HW,), jnp.float32),
          jax.ShapeDtypeStruct((NW * HW,), jnp.float32),
      ),
      mesh=mesh,
      compiler_params=pltpu.CompilerParams(
          needs_layout_passes=False, use_tc_tiling_on_sc=True),
      scratch_types=[
          pltpu.VMEM((NBUF * R, NCOL), jnp.float32),
          pltpu.VMEM((NBUF * R, NCOL), jnp.float32),
          pltpu.VMEM((HW,), jnp.float32),
          pltpu.VMEM((HW,), jnp.float32),
          pltpu.SemaphoreType.DMA,
          pltpu.SemaphoreType.DMA,
          pltpu.SemaphoreType.DMA,
          pltpu.SemaphoreType.DMA,
          pltpu.SemaphoreType.DMA,
          pltpu.SemaphoreType.DMA,
          pltpu.SemaphoreType.DMA,
          pltpu.SemaphoreType.DMA,
      ],
  )(p, t)


def _stage2_kernel(c_ref, s_ref, o_ref):
  ch = jnp.sum(c_ref[...].reshape(NW, QR, QC), axis=0)
  sh = jnp.sum(s_ref[...].reshape(NW, QR, QC), axis=0)

  colq = lax.broadcasted_iota(jnp.int32, (QC, 8), 0) // L
  qid = lax.broadcasted_iota(jnp.int32, (QC, 8), 1)
  m16 = (colq == qid).astype(jnp.float32)
  c8 = jnp.dot(ch, m16, preferred_element_type=jnp.float32)
  s8 = jnp.dot(sh, m16, preferred_element_type=jnp.float32)
  rneg = HB // 8

  npos = jnp.sum(c8[rneg:, :])
  s_pos = jnp.sum(s8[rneg:, :])
  cn = c8[:rneg, :]
  sn = s8[:rneg, :]
  nneg = jnp.sum(cn)
  s_neg = jnp.sum(sn)

  negative_num = jnp.minimum(nneg, npos * 3.0)
  k = negative_num.astype(jnp.int32)
  kf = k.astype(jnp.float32)

  q1 = lax.broadcasted_iota(jnp.int32, (8, 8), 0)
  q2 = lax.broadcasted_iota(jnp.int32, (8, 8), 1)
  u8 = (q1 > q2).astype(jnp.float32)
  r1 = lax.broadcasted_iota(jnp.int32, (rneg, rneg), 0)
  r2 = lax.broadcasted_iota(jnp.int32, (rneg, rneg), 1)
  lo = (r1 < r2).astype(jnp.float32)

  rt_c = jnp.sum(cn, axis=1, keepdims=True)
  rt_s = jnp.sum(sn, axis=1, keepdims=True)
  f = jnp.dot(lo, rt_c, preferred_element_type=jnp.float32) + jnp.dot(
      cn, u8, preferred_element_type=jnp.float32)
  g = jnp.dot(lo, rt_s, preferred_element_type=jnp.float32) + jnp.dot(
      sn, u8, preferred_element_type=jnp.float32)

  sel = (f < kf) & (f + cn >= kf)
  ratio = sn / jnp.maximum(cn, 1.0)
  part = jnp.sum(jnp.where(sel, g + (kf - f) * ratio, 0.0))
  s_topk = jnp.where(kf >= nneg, s_neg, part)

  union = npos + s_pos + s_topk
  iou = 2.0 * s_pos / union
  o_ref[0] = 1.0 - iou
  o_ref[1] = iou


def _stage2(cnt, sm):
  return pl.pallas_call(
      _stage2_kernel,
      out_shape=jax.ShapeDtypeStruct((2,), jnp.float32),
      in_specs=[
          pl.BlockSpec(memory_space=pltpu.VMEM),
          pl.BlockSpec(memory_space=pltpu.VMEM),
      ],
      out_specs=pl.BlockSpec(memory_space=pltpu.SMEM),
  )(cnt, sm)


@jax.jit
def kernel(predicted, target, training_mask):
  del training_mask
  cnt, sm = _stage1(predicted, target)
  out = _stage2(cnt.reshape(NW * QR, QC), sm.reshape(NW * QR, QC))
  return (out[0], out[1])
